# Initial kernel scaffold; baseline (speedup 1.0000x reference)
import jax
import jax.numpy as jnp
from jax.experimental import pallas as pl


def kernel(features, coords, edge_index, W1, b1, W2, b2, W3, b3):
    raise NotImplementedError("write your pallas kernel here")


# speedup vs baseline: 9.5180x; 9.5180x over previous
import functools

import jax
import jax.numpy as jnp
import numpy as np
from jax import lax
from jax.experimental import pallas as pl
from jax.experimental.pallas import tpu as pltpu
from jax.experimental.pallas import tpu_sc as plsc

NC = 2
NS = 16
NW = NC * NS
LANES = 16
CHUNK = 128


def _sigmoid(x):
    return 1.0 / (1.0 + jnp.exp(-x))


def _sc_edge_prep(NP, EPT):
    mesh = plsc.VectorSubcoreMesh(core_axis_name="c", subcore_axis_name="s")

    @functools.partial(
        pl.kernel,
        out_type=(
            jax.ShapeDtypeStruct((NW * EPT,), jnp.float32),
            jax.ShapeDtypeStruct((NW, NP), jnp.float32),
        ),
        mesh=mesh,
        scratch_types=[
            pltpu.VMEM((EPT,), jnp.int32),
            pltpu.VMEM((EPT,), jnp.int32),
            pltpu.VMEM((NP,), jnp.float32),
            pltpu.VMEM((NP,), jnp.float32),
            pltpu.VMEM((NP,), jnp.float32),
            pltpu.VMEM((NP,), jnp.float32),
            pltpu.VMEM((EPT,), jnp.float32),
        ],
        compiler_params=pltpu.CompilerParams(needs_layout_passes=False, use_tc_tiling_on_sc=False),
    )
    def k(row_hbm, col_hbm, xs_hbm, ys_hbm, zs_hbm, zeros_hbm,
          d2_hbm, cntp_hbm,
          row_v, col_v, xs_v, ys_v, zs_v, cnt_v, d2_v):
        cid = lax.axis_index("c")
        sid = lax.axis_index("s")
        wid = sid * NC + cid
        base = wid * EPT
        pltpu.sync_copy(row_hbm.at[pl.ds(base, EPT)], row_v)
        pltpu.sync_copy(col_hbm.at[pl.ds(base, EPT)], col_v)
        pltpu.sync_copy(xs_hbm, xs_v)
        pltpu.sync_copy(ys_hbm, ys_v)
        pltpu.sync_copy(zs_hbm, zs_v)
        pltpu.sync_copy(zeros_hbm, cnt_v)
        ones = jnp.full((LANES,), 1.0, jnp.float32)

        def body(i, carry):
            off = i * LANES
            r = row_v[pl.ds(off, LANES)]
            c = col_v[pl.ds(off, LANES)]
            dx = plsc.load_gather(xs_v, [r]) - plsc.load_gather(xs_v, [c])
            dy = plsc.load_gather(ys_v, [r]) - plsc.load_gather(ys_v, [c])
            dz = plsc.load_gather(zs_v, [r]) - plsc.load_gather(zs_v, [c])
            d2_v[pl.ds(off, LANES)] = dx * dx + dy * dy + dz * dz
            plsc.addupdate_scatter(cnt_v, [c], ones)
            return carry

        lax.fori_loop(0, EPT // LANES, body, 0)
        pltpu.sync_copy(d2_v, d2_hbm.at[pl.ds(base, EPT)])
        pltpu.sync_copy(cnt_v, cntp_hbm.at[wid])

    return k


def _tc_mlp(NP, NB, BLK, EP, HID, OUT_C):
    def body(d2_ref, cntp_ref, fp_ref, r_ref, qt_ref, w1_ref, b1_ref,
             w2_ref, b2_ref, w3_ref, b3t_ref, msg_ref, m3_s, c1_s):
        @pl.when(pl.program_id(0) == 0)
        def _():
            cnt1 = jnp.sum(cntp_ref[...], axis=0, keepdims=True)
            s1 = jnp.dot(cnt1, fp_ref[...],
                         preferred_element_type=jnp.float32)
            srep = jnp.dot(s1, r_ref[...],
                           preferred_element_type=jnp.float32)
            m3_s[...] = jnp.dot(qt_ref[...] * srep, w3_ref[...],
                                preferred_element_type=jnp.float32)
            c1_s[...] = jnp.dot(s1, b3t_ref[...],
                                preferred_element_type=jnp.float32)

        dd = jnp.sqrt(d2_ref[0])
        pre1 = w1_ref[...] * dd + b1_ref[...]
        h = pre1 * _sigmoid(pre1)
        pre2 = jnp.dot(w2_ref[...], h,
                       preferred_element_type=jnp.float32) + b2_ref[...]
        filt = pre2 * _sigmoid(pre2)
        msg = lax.dot_general(filt, m3_s[...], (((0,), (1,)), ((), ())),
                              preferred_element_type=jnp.float32)
        msg_ref[...] = msg + c1_s[...]

    IN_C = 16
    return pl.pallas_call(
        body,
        grid=(NB,),
        in_specs=[
            pl.BlockSpec((1, 1, BLK), lambda i: (i, 0, 0)),
            pl.BlockSpec((NW, NP), lambda i: (0, 0)),
            pl.BlockSpec((NP, IN_C), lambda i: (0, 0)),
            pl.BlockSpec((IN_C, OUT_C * IN_C), lambda i: (0, 0)),
            pl.BlockSpec((OUT_C, OUT_C * IN_C), lambda i: (0, 0)),
            pl.BlockSpec((HID, 1), lambda i: (0, 0)),
            pl.BlockSpec((HID, 1), lambda i: (0, 0)),
            pl.BlockSpec((HID, HID), lambda i: (0, 0)),
            pl.BlockSpec((HID, 1), lambda i: (0, 0)),
            pl.BlockSpec((OUT_C * IN_C, HID), lambda i: (0, 0)),
            pl.BlockSpec((IN_C, OUT_C), lambda i: (0, 0)),
        ],
        out_specs=pl.BlockSpec((BLK, OUT_C), lambda i: (i, 0)),
        out_shape=jax.ShapeDtypeStruct((EP, OUT_C), jnp.float32),
        scratch_shapes=[
            pltpu.VMEM((OUT_C, HID), jnp.float32),
            pltpu.VMEM((1, OUT_C), jnp.float32),
        ],
        compiler_params=pltpu.CompilerParams(
            dimension_semantics=("arbitrary",)),
    )


def _sc_scatter(NP, EPT, KC, OUT_C):
    mesh = plsc.VectorSubcoreMesh(core_axis_name="c", subcore_axis_name="s")
    NPS = NP // NS

    @functools.partial(
        pl.kernel,
        out_type=jax.ShapeDtypeStruct((NC, NP, OUT_C), jnp.float32),
        mesh=mesh,
        scratch_types=[
            pltpu.VMEM((EPT, OUT_C), jnp.float32),
            pltpu.VMEM((KC, CHUNK), jnp.int32),
            pltpu.VMEM_SHARED((NP, OUT_C), jnp.float32),
        ],
        compiler_params=pltpu.CompilerParams(needs_layout_passes=False, use_tc_tiling_on_sc=False),
    )
    def k(msg_hbm, idx_hbm, zeros_hbm, outp_hbm, msg_v, idx_v, acc_s):
        cid = lax.axis_index("c")
        sid = lax.axis_index("s")
        wid = sid * NC + cid
        pltpu.sync_copy(zeros_hbm.at[pl.ds(sid * NPS, NPS)],
                        acc_s.at[pl.ds(sid * NPS, NPS)])
        pltpu.sync_copy(msg_hbm.at[pl.ds(wid * EPT, EPT)], msg_v)
        pltpu.sync_copy(idx_hbm.at[wid], idx_v)
        plsc.subcore_barrier()

        def body(j, carry):
            pltpu.sync_copy(msg_v.at[pl.ds(j * CHUNK, CHUNK)],
                            acc_s.at[idx_v.at[j]], add=True)
            return carry

        lax.fori_loop(0, KC, body, 0)
        plsc.subcore_barrier()
        pltpu.sync_copy(acc_s.at[pl.ds(sid * NPS, NPS)],
                        outp_hbm.at[cid].at[pl.ds(sid * NPS, NPS)])

    return k


def _tc_combine(N, NP, OUT_C):
    def body(p_ref, o_ref):
        o_ref[...] = p_ref[0, :N, :] + p_ref[1, :N, :]

    return pl.pallas_call(
        body,
        in_specs=[pl.BlockSpec((NC, NP, OUT_C), lambda: (0, 0, 0))],
        out_specs=pl.BlockSpec((N, OUT_C), lambda: (0, 0)),
        out_shape=jax.ShapeDtypeStruct((N, OUT_C), jnp.float32),
    )


def kernel(features, coords, edge_index, W1, b1, W2, b2, W3, b3):
    N, IN_C = features.shape
    E = edge_index.shape[1]
    HID = W2.shape[0]
    OUT_C = W3.shape[0] // IN_C

    NP = ((N + 1 + 127) // 128) * 128
    EPT = -(-E // NW)
    EPT = ((EPT + 255) // 256) * 256
    EP = EPT * NW
    KC = EPT // CHUNK
    BLK = 8192
    NB = EP // BLK

    row = edge_index[0]
    col = edge_index[1]
    pad = EP - E
    row_p = jnp.concatenate([row, jnp.full((pad,), N, jnp.int32)])
    col_p = jnp.concatenate([col, jnp.full((pad,), N, jnp.int32)])
    xs = jnp.pad(coords[:, 0], (0, NP - N))
    ys = jnp.pad(coords[:, 1], (0, NP - N))
    zs = jnp.pad(coords[:, 2], (0, NP - N))
    fp = jnp.pad(features, ((0, NP - N), (0, 0)))
    zeros_np = jnp.zeros((NP,), jnp.float32)
    zeros_out = jnp.zeros((NP, OUT_C), jnp.float32)

    R = jnp.asarray(np.tile(np.eye(IN_C, dtype=np.float32), (1, OUT_C)))
    QT = jnp.asarray(np.repeat(np.eye(OUT_C, dtype=np.float32), IN_C, axis=1))
    b3rT = b3.reshape(OUT_C, IN_C).T
    b1c = b1.reshape(HID, 1)
    b2c = b2.reshape(HID, 1)

    d2, cntp = _sc_edge_prep(NP, EPT)(row_p, col_p, xs, ys, zs, zeros_np)
    d2_3d = d2.reshape(NB, 1, BLK)
    msg = _tc_mlp(NP, NB, BLK, EP, HID, OUT_C)(
        d2_3d, cntp, fp, R, QT, W1, b1c, W2, b2c, W3, b3rT)
    idx3d = row_p.reshape(NW, KC, CHUNK)
    outp = _sc_scatter(NP, EPT, KC, OUT_C)(msg, idx3d, zeros_out)
    return _tc_combine(N, NP, OUT_C)(outp)


# speedup vs baseline: 9.5925x; 1.0078x over previous
import functools

import jax
import jax.numpy as jnp
import numpy as np
from jax import lax
from jax.experimental import pallas as pl
from jax.experimental.pallas import tpu as pltpu
from jax.experimental.pallas import tpu_sc as plsc

NC = 2
NS = 16
NW = NC * NS
LANES = 16
CHUNK = 128


def _sigmoid(x):
    return 1.0 / (1.0 + jnp.exp(-x))


def _sc_edge_prep(NP, EPT):
    mesh = plsc.VectorSubcoreMesh(core_axis_name="c", subcore_axis_name="s")

    @functools.partial(
        pl.kernel,
        out_type=(
            jax.ShapeDtypeStruct((NW * EPT,), jnp.float32),
            jax.ShapeDtypeStruct((NW, NP), jnp.float32),
        ),
        mesh=mesh,
        scratch_types=[
            pltpu.VMEM((EPT,), jnp.int32),
            pltpu.VMEM((EPT,), jnp.int32),
            pltpu.VMEM((NP,), jnp.float32),
            pltpu.VMEM((NP,), jnp.float32),
            pltpu.VMEM((NP,), jnp.float32),
            pltpu.VMEM((NP,), jnp.float32),
            pltpu.VMEM((EPT,), jnp.float32),
        ],
        compiler_params=pltpu.CompilerParams(needs_layout_passes=False, use_tc_tiling_on_sc=False),
    )
    def k(row_hbm, col_hbm, xs_hbm, ys_hbm, zs_hbm, zeros_hbm,
          d2_hbm, cntp_hbm,
          row_v, col_v, xs_v, ys_v, zs_v, cnt_v, d2_v):
        cid = lax.axis_index("c")
        sid = lax.axis_index("s")
        wid = sid * NC + cid
        base = wid * EPT
        pltpu.sync_copy(row_hbm.at[pl.ds(base, EPT)], row_v)
        pltpu.sync_copy(col_hbm.at[pl.ds(base, EPT)], col_v)
        pltpu.sync_copy(xs_hbm, xs_v)
        pltpu.sync_copy(ys_hbm, ys_v)
        pltpu.sync_copy(zs_hbm, zs_v)
        pltpu.sync_copy(zeros_hbm, cnt_v)
        ones = jnp.full((LANES,), 1.0, jnp.float32)

        def body(i, carry):
            off = i * LANES
            r = row_v[pl.ds(off, LANES)]
            c = col_v[pl.ds(off, LANES)]
            dx = plsc.load_gather(xs_v, [r]) - plsc.load_gather(xs_v, [c])
            dy = plsc.load_gather(ys_v, [r]) - plsc.load_gather(ys_v, [c])
            dz = plsc.load_gather(zs_v, [r]) - plsc.load_gather(zs_v, [c])
            d2_v[pl.ds(off, LANES)] = dx * dx + dy * dy + dz * dz
            plsc.addupdate_scatter(cnt_v, [c], ones)
            return carry

        lax.fori_loop(0, EPT // LANES, body, 0)
        pltpu.sync_copy(d2_v, d2_hbm.at[pl.ds(base, EPT)])
        pltpu.sync_copy(cnt_v, cntp_hbm.at[wid])

    return k


def _tc_mlp(NP, NB, BLK, EP, HID, OUT_C):
    def body(d2_ref, cntp_ref, fp_ref, r_ref, qt_ref, w1_ref, b1_ref,
             w2_ref, b2_ref, w3_ref, b3t_ref, msg_ref, m3_s, c1_s):
        @pl.when(pl.program_id(0) == 0)
        def _():
            cnt1 = jnp.sum(cntp_ref[...], axis=0, keepdims=True)
            s1 = jnp.dot(cnt1, fp_ref[...],
                         preferred_element_type=jnp.float32)
            srep = jnp.dot(s1, r_ref[...],
                           preferred_element_type=jnp.float32)
            m3_s[...] = jnp.dot(qt_ref[...] * srep, w3_ref[...],
                                preferred_element_type=jnp.float32)
            c1_s[...] = jnp.dot(s1, b3t_ref[...],
                                preferred_element_type=jnp.float32)

        dd = jnp.sqrt(d2_ref[0])
        pre1 = w1_ref[...] * dd + b1_ref[...]
        h = pre1 * _sigmoid(pre1)
        pre2 = jnp.dot(w2_ref[...], h,
                       preferred_element_type=jnp.float32) + b2_ref[...]
        filt = pre2 * _sigmoid(pre2)
        msg = lax.dot_general(filt, m3_s[...], (((0,), (1,)), ((), ())),
                              preferred_element_type=jnp.float32)
        msg_ref[...] = msg + c1_s[...]

    IN_C = 16
    return pl.pallas_call(
        body,
        grid=(NB,),
        in_specs=[
            pl.BlockSpec((1, 1, BLK), lambda i: (i, 0, 0)),
            pl.BlockSpec((NW, NP), lambda i: (0, 0)),
            pl.BlockSpec((NP, IN_C), lambda i: (0, 0)),
            pl.BlockSpec((IN_C, OUT_C * IN_C), lambda i: (0, 0)),
            pl.BlockSpec((OUT_C, OUT_C * IN_C), lambda i: (0, 0)),
            pl.BlockSpec((HID, 1), lambda i: (0, 0)),
            pl.BlockSpec((HID, 1), lambda i: (0, 0)),
            pl.BlockSpec((HID, HID), lambda i: (0, 0)),
            pl.BlockSpec((HID, 1), lambda i: (0, 0)),
            pl.BlockSpec((OUT_C * IN_C, HID), lambda i: (0, 0)),
            pl.BlockSpec((IN_C, OUT_C), lambda i: (0, 0)),
        ],
        out_specs=pl.BlockSpec((BLK, OUT_C), lambda i: (i, 0)),
        out_shape=jax.ShapeDtypeStruct((EP, OUT_C), jnp.float32),
        scratch_shapes=[
            pltpu.VMEM((OUT_C, HID), jnp.float32),
            pltpu.VMEM((1, OUT_C), jnp.float32),
        ],
        compiler_params=pltpu.CompilerParams(
            dimension_semantics=("arbitrary",),
            fuse_transposed_lhs_in_matmul=True),
    )


def _sc_scatter(NP, EPT, KC, OUT_C):
    mesh = plsc.VectorSubcoreMesh(core_axis_name="c", subcore_axis_name="s")
    NPS = NP // NS

    @functools.partial(
        pl.kernel,
        out_type=jax.ShapeDtypeStruct((NC, NP, OUT_C), jnp.float32),
        mesh=mesh,
        scratch_types=[
            pltpu.VMEM((EPT, OUT_C), jnp.float32),
            pltpu.VMEM((KC, CHUNK), jnp.int32),
            pltpu.VMEM_SHARED((NP, OUT_C), jnp.float32),
        ],
        compiler_params=pltpu.CompilerParams(needs_layout_passes=False, use_tc_tiling_on_sc=False),
    )
    def k(msg_hbm, idx_hbm, zeros_hbm, outp_hbm, msg_v, idx_v, acc_s):
        cid = lax.axis_index("c")
        sid = lax.axis_index("s")
        wid = sid * NC + cid
        pltpu.sync_copy(zeros_hbm.at[pl.ds(sid * NPS, NPS)],
                        acc_s.at[pl.ds(sid * NPS, NPS)])
        pltpu.sync_copy(msg_hbm.at[pl.ds(wid * EPT, EPT)], msg_v)
        pltpu.sync_copy(idx_hbm.at[wid], idx_v)
        plsc.subcore_barrier()

        def body(j, carry):
            pltpu.sync_copy(msg_v.at[pl.ds(j * CHUNK, CHUNK)],
                            acc_s.at[idx_v.at[j]], add=True)
            return carry

        lax.fori_loop(0, KC, body, 0)
        plsc.subcore_barrier()
        pltpu.sync_copy(acc_s.at[pl.ds(sid * NPS, NPS)],
                        outp_hbm.at[cid].at[pl.ds(sid * NPS, NPS)])

    return k


def _tc_combine(N, NP, OUT_C):
    def body(p_ref, o_ref):
        o_ref[...] = p_ref[0, :N, :] + p_ref[1, :N, :]

    return pl.pallas_call(
        body,
        in_specs=[pl.BlockSpec((NC, NP, OUT_C), lambda: (0, 0, 0))],
        out_specs=pl.BlockSpec((N, OUT_C), lambda: (0, 0)),
        out_shape=jax.ShapeDtypeStruct((N, OUT_C), jnp.float32),
    )


def kernel(features, coords, edge_index, W1, b1, W2, b2, W3, b3):
    N, IN_C = features.shape
    E = edge_index.shape[1]
    HID = W2.shape[0]
    OUT_C = W3.shape[0] // IN_C

    NP = ((N + 1 + 127) // 128) * 128
    EPT = -(-E // NW)
    EPT = ((EPT + 255) // 256) * 256
    EP = EPT * NW
    KC = EPT // CHUNK
    BLK = 16384
    NB = EP // BLK

    row = edge_index[0]
    col = edge_index[1]
    pad = EP - E
    row_p = jnp.concatenate([row, jnp.full((pad,), N, jnp.int32)])
    col_p = jnp.concatenate([col, jnp.full((pad,), N, jnp.int32)])
    xs = jnp.pad(coords[:, 0], (0, NP - N))
    ys = jnp.pad(coords[:, 1], (0, NP - N))
    zs = jnp.pad(coords[:, 2], (0, NP - N))
    fp = jnp.pad(features, ((0, NP - N), (0, 0)))
    zeros_np = jnp.zeros((NP,), jnp.float32)
    zeros_out = jnp.zeros((NP, OUT_C), jnp.float32)

    R = jnp.asarray(np.tile(np.eye(IN_C, dtype=np.float32), (1, OUT_C)))
    QT = jnp.asarray(np.repeat(np.eye(OUT_C, dtype=np.float32), IN_C, axis=1))
    b3rT = b3.reshape(OUT_C, IN_C).T
    b1c = b1.reshape(HID, 1)
    b2c = b2.reshape(HID, 1)

    d2, cntp = _sc_edge_prep(NP, EPT)(row_p, col_p, xs, ys, zs, zeros_np)
    d2_3d = d2.reshape(NB, 1, BLK)
    msg = _tc_mlp(NP, NB, BLK, EP, HID, OUT_C)(
        d2_3d, cntp, fp, R, QT, W1, b1c, W2, b2c, W3, b3rT)
    idx3d = row_p.reshape(NW, KC, CHUNK)
    outp = _sc_scatter(NP, EPT, KC, OUT_C)(msg, idx3d, zeros_out)
    return _tc_combine(N, NP, OUT_C)(outp)
